# baseline (speedup 1.0000x reference)
import functools

import jax
import jax.numpy as jnp
from jax import lax
from jax.experimental import pallas as pl

_NUM_KPS = 17
_POS_DISTANCE = 4.0
_ROI_EXPAND = 1.2
_FW = 56
_FH = 56
_HW = _FW * _FH


def _argmax_body(rois_ref, s_ref, ms_ref, bx_ref, by_ref, scx_ref, scy_ref,
                 idx_ref):
    s = s_ref[...]
    m = jnp.max(s, axis=(-2, -1))
    ir = lax.broadcasted_iota(jnp.int32, s.shape, 2)
    ic = lax.broadcasted_iota(jnp.int32, s.shape, 3)
    fid = ir * _FW + ic
    fidx = jnp.min(jnp.where(s == m[..., None, None], fid, _HW), axis=(-2, -1))
    idx_ref[...] = fidx
    fidxf = fidx.astype(jnp.float32)
    iy = jnp.floor(fidxf / _FW)
    ix = fidxf - iy * _FW

    r = rois_ref[...]
    w = (r[:, 2] - r[:, 0]) * _ROI_EXPAND
    h = (r[:, 3] - r[:, 1]) * _ROI_EXPAND
    x1 = (r[:, 2] + r[:, 0]) * 0.5 - w * 0.5
    y1 = (r[:, 3] + r[:, 1]) * 0.5 - h * 0.5
    sx = _FW / (w + 1.0)
    sy = _FW / (h + 1.0)

    ms_ref[...] = m
    bx_ref[...] = ix / sx[:, None] + x1[:, None]
    by_ref[...] = iy / sy[:, None] + y1[:, None]
    scx_ref[...] = jnp.broadcast_to((_POS_DISTANCE / sx)[:, None], fidx.shape)
    scy_ref[...] = jnp.broadcast_to((_POS_DISTANCE / sy)[:, None], fidx.shape)


def _select_body(d_ref, idx_ref, bx_ref, by_ref, scx_ref, scy_ref,
                 px_ref, py_ref):
    fidx = idx_ref[...]
    shape4 = (fidx.shape[0], fidx.shape[1], _FW, _FH)
    ir = lax.broadcasted_iota(jnp.int32, shape4, 2)
    ic = lax.broadcasted_iota(jnp.int32, shape4, 3)
    onehot = (ir * _FW + ic) == fidx[..., None, None]
    dx = jnp.sum(jnp.where(onehot, d_ref[:, :, 0], 0.0), axis=(-2, -1))
    dy = jnp.sum(jnp.where(onehot, d_ref[:, :, 1], 0.0), axis=(-2, -1))
    px_ref[...] = bx_ref[...] + dx * scx_ref[...]
    py_ref[...] = by_ref[...] + dy * scy_ref[...]


@functools.partial(jax.jit, static_argnames=("block",))
def kernel(batch_rois, kps_rcnn_cls_pred, kps_rcnn_reg_pred, block=16):
    bs, r_per = batch_rois.shape[0], batch_rois.shape[1]
    n = bs * r_per
    scores = kps_rcnn_cls_pred.reshape(n, _NUM_KPS, _FW, _FH)
    deltas = kps_rcnn_reg_pred.reshape(n, _NUM_KPS, 2, _FW, _FH)
    rois = batch_rois[..., :4].reshape(n, 4)

    grid = (n // block,)
    spec2d = pl.BlockSpec((block, _NUM_KPS), lambda i: (i, 0))
    out2d_f = jax.ShapeDtypeStruct((n, _NUM_KPS), jnp.float32)
    out2d_i = jax.ShapeDtypeStruct((n, _NUM_KPS), jnp.int32)

    ms, bx, by, scx, scy, fidx = pl.pallas_call(
        _argmax_body,
        grid=grid,
        in_specs=[
            pl.BlockSpec((block, 4), lambda i: (i, 0)),
            pl.BlockSpec((block, _NUM_KPS, _FW, _FH), lambda i: (i, 0, 0, 0)),
        ],
        out_specs=[spec2d] * 6,
        out_shape=[out2d_f] * 5 + [out2d_i],
    )(rois, scores)

    px, py = pl.pallas_call(
        _select_body,
        grid=grid,
        in_specs=[
            pl.BlockSpec((block, _NUM_KPS, 2, _FW, _FH),
                         lambda i: (i, 0, 0, 0, 0)),
            spec2d, spec2d, spec2d, spec2d, spec2d,
        ],
        out_specs=[spec2d] * 2,
        out_shape=[out2d_f] * 2,
    )(deltas, fidx, bx, by, scx, scy)

    return jnp.stack([px, py, ms], axis=-1).reshape(bs, r_per, _NUM_KPS, 3)


# speedup vs baseline: 1.0606x; 1.0606x over previous
import functools

import jax
import jax.numpy as jnp
from jax import lax
from jax.experimental import pallas as pl

_NUM_KPS = 17
_POS_DISTANCE = 4.0
_ROI_EXPAND = 1.2
_FW = 56
_FH = 56
_HW = _FW * _FH


def _decode_block(rois, s, d, px_ref, py_ref, ms_ref):
    m = jnp.max(s, axis=(-2, -1))
    ir = lax.broadcasted_iota(jnp.int32, s.shape, 2)
    ic = lax.broadcasted_iota(jnp.int32, s.shape, 3)
    fid = ir * _FW + ic
    fidx = jnp.min(jnp.where(s == m[..., None, None], fid, _HW), axis=(-2, -1))
    onehot = fid == fidx[..., None, None]
    dx = jnp.sum(jnp.where(onehot, d[:, :, 0], 0.0), axis=(-2, -1))
    dy = jnp.sum(jnp.where(onehot, d[:, :, 1], 0.0), axis=(-2, -1))
    fidxf = fidx.astype(jnp.float32)
    iy = jnp.floor(fidxf / _FW)
    ix = fidxf - iy * _FW

    w = (rois[:, 2] - rois[:, 0]) * _ROI_EXPAND
    h = (rois[:, 3] - rois[:, 1]) * _ROI_EXPAND
    x1 = (rois[:, 2] + rois[:, 0]) * 0.5 - w * 0.5
    y1 = (rois[:, 3] + rois[:, 1]) * 0.5 - h * 0.5
    sx = _FW / (w + 1.0)
    sy = _FW / (h + 1.0)

    px_ref[...] = (ix + dx * _POS_DISTANCE) / sx[:, None] + x1[:, None]
    py_ref[...] = (iy + dy * _POS_DISTANCE) / sy[:, None] + y1[:, None]
    ms_ref[...] = m


def _decode_body(rois_a, s_a, d_a, rois_b, s_b, d_b,
                 px_a, py_a, ms_a, px_b, py_b, ms_b):
    _decode_block(rois_a[...], s_a[...], d_a[...], px_a, py_a, ms_a)
    _decode_block(rois_b[...], s_b[...], d_b[...], px_b, py_b, ms_b)


@functools.partial(jax.jit, static_argnames=("block",))
def kernel(batch_rois, kps_rcnn_cls_pred, kps_rcnn_reg_pred, block=8):
    bs, r_per = batch_rois.shape[0], batch_rois.shape[1]
    n = bs * r_per
    half = n // 2
    scores = kps_rcnn_cls_pred.reshape(n, _NUM_KPS, _FW, _FH)
    deltas = kps_rcnn_reg_pred.reshape(n, _NUM_KPS, 2, _FW, _FH)
    rois = batch_rois[..., :4].reshape(n, 4)

    nsteps = half // block
    spec_rois_a = pl.BlockSpec((block, 4), lambda i: (i, 0))
    spec_rois_b = pl.BlockSpec((block, 4), lambda i: (i + nsteps, 0))
    spec_s_a = pl.BlockSpec((block, _NUM_KPS, _FW, _FH),
                            lambda i: (i, 0, 0, 0))
    spec_s_b = pl.BlockSpec((block, _NUM_KPS, _FW, _FH),
                            lambda i: (i + nsteps, 0, 0, 0))
    spec_d_a = pl.BlockSpec((block, _NUM_KPS, 2, _FW, _FH),
                            lambda i: (i, 0, 0, 0, 0))
    spec_d_b = pl.BlockSpec((block, _NUM_KPS, 2, _FW, _FH),
                            lambda i: (i + nsteps, 0, 0, 0, 0))
    spec2d = pl.BlockSpec((block, _NUM_KPS), lambda i: (i, 0))
    out2d = jax.ShapeDtypeStruct((half, _NUM_KPS), jnp.float32)

    px_a, py_a, ms_a, px_b, py_b, ms_b = pl.pallas_call(
        _decode_body,
        grid=(nsteps,),
        in_specs=[spec_rois_a, spec_s_a, spec_d_a,
                  spec_rois_b, spec_s_b, spec_d_b],
        out_specs=[spec2d] * 6,
        out_shape=[out2d] * 6,
    )(rois, scores, deltas, rois, scores, deltas)

    px = jnp.concatenate([px_a, px_b])
    py = jnp.concatenate([py_a, py_b])
    ms = jnp.concatenate([ms_a, ms_b])
    return jnp.stack([px, py, ms], axis=-1).reshape(bs, r_per, _NUM_KPS, 3)


# speedup vs baseline: 1.0807x; 1.0190x over previous
import functools

import jax
import jax.numpy as jnp
from jax import lax
from jax.experimental import pallas as pl

_NUM_KPS = 17
_POS_DISTANCE = 4.0
_ROI_EXPAND = 1.2
_FW = 56
_FH = 56
_HW = _FW * _FH


def _decode_body(rois_ref, s_ref, d_ref, px_ref, py_ref, ms_ref):
    s = s_ref[...]
    m = jnp.max(s, axis=(-2, -1))
    ir = lax.broadcasted_iota(jnp.int32, s.shape, 2)
    ic = lax.broadcasted_iota(jnp.int32, s.shape, 3)
    fid = ir * _FW + ic
    fidx = jnp.min(jnp.where(s == m[..., None, None], fid, _HW), axis=(-2, -1))
    onehot = fid == fidx[..., None, None]
    dx = jnp.sum(jnp.where(onehot, d_ref[:, :, 0], 0.0), axis=(-2, -1))
    dy = jnp.sum(jnp.where(onehot, d_ref[:, :, 1], 0.0), axis=(-2, -1))
    fidxf = fidx.astype(jnp.float32)
    iy = jnp.floor(fidxf / _FW)
    ix = fidxf - iy * _FW

    r = rois_ref[...]
    w = (r[:, 2] - r[:, 0]) * _ROI_EXPAND
    h = (r[:, 3] - r[:, 1]) * _ROI_EXPAND
    x1 = (r[:, 2] + r[:, 0]) * 0.5 - w * 0.5
    y1 = (r[:, 3] + r[:, 1]) * 0.5 - h * 0.5
    sx = _FW / (w + 1.0)
    sy = _FW / (h + 1.0)

    px_ref[...] = (ix + dx * _POS_DISTANCE) / sx[:, None] + x1[:, None]
    py_ref[...] = (iy + dy * _POS_DISTANCE) / sy[:, None] + y1[:, None]
    ms_ref[...] = m


@functools.partial(jax.jit, static_argnames=("block",))
def kernel(batch_rois, kps_rcnn_cls_pred, kps_rcnn_reg_pred, block=16):
    bs, r_per = batch_rois.shape[0], batch_rois.shape[1]
    n = bs * r_per
    scores = kps_rcnn_cls_pred.reshape(n, _NUM_KPS, _FW, _FH)
    deltas = kps_rcnn_reg_pred.reshape(n, _NUM_KPS, 2, _FW, _FH)
    rois = batch_rois[..., :4].reshape(n, 4)

    grid = (n // block,)
    spec2d = pl.BlockSpec((block, _NUM_KPS), lambda i: (i, 0))
    out2d = jax.ShapeDtypeStruct((n, _NUM_KPS), jnp.float32)
    px, py, ms = pl.pallas_call(
        _decode_body,
        grid=grid,
        in_specs=[
            pl.BlockSpec((block, 4), lambda i: (i, 0)),
            pl.BlockSpec((block, _NUM_KPS, _FW, _FH), lambda i: (i, 0, 0, 0)),
            pl.BlockSpec((block, _NUM_KPS, 2, _FW, _FH),
                         lambda i: (i, 0, 0, 0, 0)),
        ],
        out_specs=[spec2d] * 3,
        out_shape=[out2d] * 3,
    )(rois, scores, deltas)

    return jnp.stack([px, py, ms], axis=-1).reshape(bs, r_per, _NUM_KPS, 3)
